# baseline (speedup 1.0000x reference)
import functools

import jax
import jax.numpy as jnp
from jax.experimental import pallas as pl
from jax.experimental.pallas import tpu as pltpu

NUM_GRAPHS = 64
EPS = 1e-5


def _rowmm(xs, Ws, b, adds=(), res=None, relu=False, blk=4000):
    n = xs[0].shape[0]
    M = Ws[0].shape[1]
    Wcat = jnp.concatenate(Ws, axis=0)
    K = Wcat.shape[0]
    b8 = jnp.tile(b.reshape(1, M), (8, 1))
    n_x = len(xs)
    n_a = len(adds)
    has_res = res is not None
    grid = n // blk

    def body(*refs):
        xr = refs[:n_x]
        Wr = refs[n_x]
        br = refs[n_x + 1]
        ar = refs[n_x + 2:n_x + 2 + n_a]
        rr = refs[n_x + 2 + n_a] if has_res else None
        out = refs[-1]
        k0 = 0
        acc = None
        for i, x in enumerate(xr):
            k = x.shape[1]
            p = jnp.dot(x[...], Wr[pl.ds(k0, k), :],
                        preferred_element_type=jnp.float32)
            acc = p if acc is None else acc + p
            k0 += k
        acc = acc + br[0:1, :]
        for a in ar:
            acc = acc + a[...]
        if relu:
            acc = jnp.maximum(acc, 0.0)
        if has_res:
            acc = acc + rr[...]
        out[...] = acc

    in_specs = [pl.BlockSpec((blk, x.shape[1]), lambda i: (i, 0)) for x in xs]
    in_specs.append(pl.BlockSpec((K, M), lambda i: (0, 0)))
    in_specs.append(pl.BlockSpec((8, M), lambda i: (0, 0)))
    in_specs += [pl.BlockSpec((blk, M), lambda i: (i, 0)) for _ in adds]
    operands = list(xs) + [Wcat, b8] + list(adds)
    if has_res:
        in_specs.append(pl.BlockSpec((blk, M), lambda i: (i, 0)))
        operands.append(res)
    return pl.pallas_call(
        body,
        grid=(grid,),
        in_specs=in_specs,
        out_specs=pl.BlockSpec((blk, M), lambda i: (i, 0)),
        out_shape=jax.ShapeDtypeStruct((n, M), jnp.float32),
    )(*operands)


def _graph_norm(v, batch, w, bias):
    F = v.shape[-1]
    cnt = jax.ops.segment_sum(jnp.ones((v.shape[0],), jnp.float32), batch,
                              num_segments=NUM_GRAPHS)
    denom = jnp.maximum(cnt * F, 1.0)
    mean = jax.ops.segment_sum(v.sum(-1), batch, num_segments=NUM_GRAPHS) / denom
    var = (jax.ops.segment_sum((v * v).sum(-1), batch, num_segments=NUM_GRAPHS)
           / denom - mean * mean)
    inv = 1.0 / jnp.sqrt(var + EPS)
    return (v - mean[batch][:, None]) * inv[batch][:, None] * w + bias


def kernel(x, edge_attr, angle_attr, params, node_batch, edge_index,
           edge_batch, threebody_index, angle_batch):
    D = x.shape[1]
    ED = edge_attr.shape[1]
    AD = angle_attr.shape[1]
    src_e, dst_e = edge_index[0], edge_index[1]
    src_t, dst_t = threebody_index[0], threebody_index[1]
    e = edge_attr
    a = angle_attr
    for lp in params['layers']:
        Wa, Wb, Wc = lp['ne_W'][:D], lp['ne_W'][D:2 * D], lp['ne_W'][2 * D:]
        A, B, C = lp['ea_W'][:ED], lp['ea_W'][ED:2 * ED], lp['ea_W'][2 * ED:]
        Wm1, Wm2 = lp['emp_Wm'][:ED], lp['emp_Wm'][ED:]
        Wu1, Wu2 = lp['emp_Wu'][:ED], lp['emp_Wu'][ED:]
        Nm1, Nm2 = lp['nmp_Wm'][:D], lp['nmp_Wm'][D:]
        Nu1, Nu2 = lp['nmp_Wu'][:D], lp['nmp_Wu'][D:]

        z160 = jnp.zeros((160,), jnp.float32)
        proj = _rowmm([x], [jnp.concatenate([Wa, Wb, Nm1], axis=1)], z160,
                      blk=1000)
        xa, xb, xm = proj[:, :ED], proj[:, ED:2 * ED], proj[:, 2 * ED:]

        g1 = xa[src_e] + xb[dst_e]
        e_pre = _rowmm([e], [Wc], lp['ne_b'], adds=(g1,), res=e)
        e1 = _graph_norm(e_pre, edge_batch, lp['en_w'], lp['en_b'])

        z48 = jnp.zeros((48,), jnp.float32)
        ep = _rowmm([e1], [jnp.concatenate([A, B, Wm1], axis=1)], z48)
        ea, eb, em = ep[:, :ED], ep[:, ED:2 * ED], ep[:, 2 * ED:]

        g2 = ea[src_t] + eb[dst_t]
        a_pre = _rowmm([a], [C], lp['ea_b'], adds=(g2,), res=a)
        a1 = _graph_norm(a_pre, angle_batch, lp['an_w'], lp['an_b'])

        g3 = em[src_t]
        m = _rowmm([a1], [Wm2], lp['emp_bm'], adds=(g3,), relu=True)
        agg_e = jax.ops.segment_sum(m, dst_t, num_segments=e.shape[0])
        e2 = _rowmm([e1, agg_e], [Wu1, Wu2], lp['emp_bu'], relu=True, res=e1)

        g4 = xm[src_e]
        m2 = _rowmm([e2], [Nm2], lp['nmp_bm'], adds=(g4,), relu=True)
        agg_n = jax.ops.segment_sum(m2, dst_e, num_segments=x.shape[0])
        x = _rowmm([x, agg_n], [Nu1, Nu2], lp['nmp_bu'], relu=True, res=x,
                   blk=1000)
        e = e2
        a = a1
    return x


# speedup vs baseline: 6.4992x; 6.4992x over previous
import functools

import jax
import jax.numpy as jnp
from jax import lax
from jax.experimental import pallas as pl
from jax.experimental.pallas import tpu as pltpu
from jax.experimental.pallas import tpu_sc as plsc

NUM_GRAPHS = 64
EPS = 1e-5

_MESH = plsc.VectorSubcoreMesh(core_axis_name="c", subcore_axis_name="s")
_NOTILE = pltpu.CompilerParams(use_tc_tiling_on_sc=False)
_NW = 32


def _sc_gather(table, idx2, chr_):
    V, D = table.shape
    R = idx2.shape[0]
    N = R * 128
    che = chr_ * 128
    nch = R // chr_
    npw = -(-nch // _NW)

    @functools.partial(
        pl.kernel, mesh=_MESH,
        out_type=jax.ShapeDtypeStruct((N, D), jnp.float32),
        scratch_types=[
            pltpu.VMEM((chr_, 128), jnp.int32),
            pltpu.VMEM((che, D), jnp.float32),
            pltpu.SemaphoreType.DMA,
        ],
        compiler_params=_NOTILE,
    )
    def k(tab, idx, out, idx_v, buf_v, sem):
        wid = lax.axis_index("s") * 2 + lax.axis_index("c")

        def body(i, carry):
            ch = i * _NW + wid

            @pl.when(ch < nch)
            def _():
                pltpu.sync_copy(idx.at[pl.ds(ch * chr_, chr_)], idx_v)
                cps = [pltpu.async_copy(tab.at[idx_v.at[j]],
                                        buf_v.at[pl.ds(j * 128, 128)], sem)
                       for j in range(chr_)]
                for c in cps:
                    c.wait()
                pltpu.sync_copy(buf_v, out.at[pl.ds(ch * che, che)])
            return carry

        lax.fori_loop(0, npw, body, 0)

    return k(table, idx2)


def _rowmm(xs, Ws, b, adds=(), res=None, relu=False, blk=4000):
    n = xs[0].shape[0]
    M = Ws[0].shape[1]
    Wcat = jnp.concatenate(Ws, axis=0)
    K = Wcat.shape[0]
    b8 = jnp.tile(b.reshape(1, M), (8, 1))
    n_x = len(xs)
    n_a = len(adds)
    has_res = res is not None
    grid = n // blk

    def body(*refs):
        xr = refs[:n_x]
        Wr = refs[n_x]
        br = refs[n_x + 1]
        ar = refs[n_x + 2:n_x + 2 + n_a]
        rr = refs[n_x + 2 + n_a] if has_res else None
        out = refs[-1]
        k0 = 0
        acc = None
        for x in xr:
            k = x.shape[1]
            p = jnp.dot(x[...], Wr[pl.ds(k0, k), :],
                        preferred_element_type=jnp.float32)
            acc = p if acc is None else acc + p
            k0 += k
        acc = acc + br[0:1, :]
        for a, col in zip(ar, add_cols):
            acc = acc + a[...][:, col * M:(col + 1) * M]
        if relu:
            acc = jnp.maximum(acc, 0.0)
        if has_res:
            acc = acc + rr[...]
        out[...] = acc

    in_specs = [pl.BlockSpec((blk, x.shape[1]), lambda i: (i, 0)) for x in xs]
    in_specs.append(pl.BlockSpec((K, M), lambda i: (0, 0)))
    in_specs.append(pl.BlockSpec((8, M), lambda i: (0, 0)))
    operands = list(xs) + [Wcat, b8]
    add_cols = []
    for a in adds:
        arr, col = a if isinstance(a, tuple) else (a, 0)
        in_specs.append(pl.BlockSpec((blk, arr.shape[1]), lambda i: (i, 0)))
        operands.append(arr)
        add_cols.append(col)
    if has_res:
        in_specs.append(pl.BlockSpec((blk, M), lambda i: (i, 0)))
        operands.append(res)
    return pl.pallas_call(
        body,
        grid=(grid,),
        in_specs=in_specs,
        out_specs=pl.BlockSpec((blk, M), lambda i: (i, 0)),
        out_shape=jax.ShapeDtypeStruct((n, M), jnp.float32),
    )(*operands)


def _graph_norm(v, batch, w, bias):
    F = v.shape[-1]
    cnt = jax.ops.segment_sum(jnp.ones((v.shape[0],), jnp.float32), batch,
                              num_segments=NUM_GRAPHS)
    denom = jnp.maximum(cnt * F, 1.0)
    mean = jax.ops.segment_sum(v.sum(-1), batch, num_segments=NUM_GRAPHS) / denom
    var = (jax.ops.segment_sum((v * v).sum(-1), batch, num_segments=NUM_GRAPHS)
           / denom - mean * mean)
    inv = 1.0 / jnp.sqrt(var + EPS)
    return (v - mean[batch][:, None]) * inv[batch][:, None] * w + bias


def kernel(x, edge_attr, angle_attr, params, node_batch, edge_index,
           edge_batch, threebody_index, angle_batch):
    D = x.shape[1]
    ED = edge_attr.shape[1]
    N_E = edge_attr.shape[0]
    N_T = angle_attr.shape[0]
    src_e2 = edge_index[0].reshape(N_E // 128, 128)
    dst_e2 = edge_index[1].reshape(N_E // 128, 128)
    src_t2 = threebody_index[0].reshape(N_T // 128, 128)
    dst_t2 = threebody_index[1].reshape(N_T // 128, 128)
    src_e, dst_e = edge_index[0], edge_index[1]
    dst_t = threebody_index[1]
    e = edge_attr
    a = angle_attr
    for lp in params['layers']:
        Wa, Wb, Wc = lp['ne_W'][:D], lp['ne_W'][D:2 * D], lp['ne_W'][2 * D:]
        A, B, C = lp['ea_W'][:ED], lp['ea_W'][ED:2 * ED], lp['ea_W'][2 * ED:]
        Wm1, Wm2 = lp['emp_Wm'][:ED], lp['emp_Wm'][ED:]
        Wu1, Wu2 = lp['emp_Wu'][:ED], lp['emp_Wu'][ED:]
        Nm1, Nm2 = lp['nmp_Wm'][:D], lp['nmp_Wm'][D:]
        Nu1, Nu2 = lp['nmp_Wu'][:D], lp['nmp_Wu'][D:]

        xab = _rowmm([x], [jnp.concatenate([Wa, Wb], axis=1)],
                     jnp.zeros((32,), jnp.float32), blk=1000)
        xm = _rowmm([x], [Nm1], jnp.zeros((128,), jnp.float32), blk=1000)

        ga = _sc_gather(xab[:, :ED], src_e2, 10)
        gb = _sc_gather(xab[:, ED:], dst_e2, 10)
        e_pre = _rowmm([e], [Wc], lp['ne_b'], adds=(ga, gb), res=e)
        e1 = _graph_norm(e_pre, edge_batch, lp['en_w'], lp['en_b'])

        epA = _rowmm([e1], [jnp.concatenate([A, Wm1], axis=1)],
                     jnp.zeros((32,), jnp.float32))
        epB = _rowmm([e1], [B], jnp.zeros((ED,), jnp.float32))

        gAM = _sc_gather(epA, src_t2, 10)
        gB = _sc_gather(epB, dst_t2, 10)

        a_pre = _rowmm([a], [C], lp['ea_b'], adds=((gAM, 0), gB), res=a)
        a1 = _graph_norm(a_pre, angle_batch, lp['an_w'], lp['an_b'])

        m = _rowmm([a1], [Wm2], lp['emp_bm'], adds=((gAM, 1),), relu=True)
        agg_e = jax.ops.segment_sum(m, dst_t, num_segments=N_E)
        e2 = _rowmm([e1, agg_e], [Wu1, Wu2], lp['emp_bu'], relu=True, res=e1)

        g4 = _sc_gather(xm, src_e2, 4)
        m2 = _rowmm([e2], [Nm2], lp['nmp_bm'], adds=(g4,), relu=True)
        agg_n = jax.ops.segment_sum(m2, dst_e, num_segments=x.shape[0])
        x = _rowmm([x, agg_n], [Nu1, Nu2], lp['nmp_bu'], relu=True, res=x,
                   blk=1000)
        e = e2
        a = a1
    return x
